# baseline (speedup 1.0000x reference)
import functools

import jax
import jax.numpy as jnp
from jax import lax
from jax.experimental import pallas as pl
from jax.experimental.pallas import tpu as pltpu
from jax.experimental.pallas import tpu_sc as plsc

N = 200000
C = 81
BG = 80
EXCL = 79
NC = 2
NS = 16
NW = NC * NS
CHUNK = N // NW
BLK = 624
NBLK = CHUNK // BLK
TAIL = CHUNK - NBLK * BLK
GRP = BLK // 16
IMAX = 2**31 - 1
NEG_INF = float("-inf")


def _merge_top3(cv, ci, bv, bi):
    lanes = lax.iota(jnp.int32, 16)
    nv = jnp.full((16,), NEG_INF, jnp.float32)
    ni = jnp.full((16,), IMAX, jnp.int32)
    av, ai = bv, bi
    t3 = NEG_INF
    for k in range(3):
        m = jnp.maximum(jnp.max(av), jnp.max(cv))
        ia = jnp.min(jnp.where(av == m, ai, IMAX))
        ic = jnp.min(jnp.where(cv == m, ci, IMAX))
        i = jnp.minimum(ia, ic)
        nv = jnp.where(lanes == k, m, nv)
        ni = jnp.where(lanes == k, i, ni)
        hit_a = ai == i
        av = jnp.where(hit_a, NEG_INF, av)
        ai = jnp.where(hit_a, IMAX, ai)
        hit_c = ci == i
        cv = jnp.where(hit_c, NEG_INF, cv)
        ci = jnp.where(hit_c, IMAX, ci)
        t3 = m
    return nv, ni, t3


def _row_max(buf, rid):
    base = rid * C
    m = jnp.full((16,), NEG_INF, jnp.float32)
    for c in range(C):
        if c == EXCL:
            continue
        m = jnp.maximum(m, plsc.load_gather(buf, [base + c]))
    return m


def _phase1_body(scores_hbm, labels_hbm, out_v, out_i, out_lab, out_cnt,
                 buf, labv, ov, oi, ol, oc):
    w = lax.axis_index("s") * NC + lax.axis_index("c")
    row0 = w * CHUNK
    lanes = lax.iota(jnp.int32, 16)

    pltpu.sync_copy(labels_hbm.at[w], labv.at[pl.ds(0, CHUNK)])

    def group_step(g_off, carry):
        bv, bi, t3, acc, blk_base = carry
        rid = g_off + lanes
        mrow = _row_max(buf, rid)
        lv = labv[pl.ds(blk_base + g_off, 16)]
        fg = lv != BG
        metric = jnp.where(fg, -mrow, NEG_INF)
        gidx = row0 + blk_base + rid
        acc = acc + jnp.where(fg, 1, 0).astype(jnp.int32)
        gmax = jnp.max(metric)

        def do_merge(args):
            bv_, bi_, _ = args
            return _merge_top3(metric, gidx, bv_, bi_)

        bv, bi, t3 = lax.cond(gmax >= t3, do_merge, lambda a: a, (bv, bi, t3))
        return bv, bi, t3, acc, blk_base

    def block_step(b, carry):
        bv, bi, t3, acc = carry
        pltpu.sync_copy(scores_hbm.at[w, pl.ds(b * (BLK * C), BLK * C)], buf)
        blk_base = b * BLK

        def grp(g, c):
            r = group_step(g * 16, c)
            return r

        bv, bi, t3, acc, _ = lax.fori_loop(
            0, GRP, grp, (bv, bi, t3, acc, blk_base))
        return bv, bi, t3, acc

    bv0 = jnp.full((16,), NEG_INF, jnp.float32)
    bi0 = jnp.full((16,), IMAX, jnp.int32)
    acc0 = jnp.zeros((16,), jnp.int32)
    bv, bi, t3, acc = lax.fori_loop(
        0, NBLK, block_step, (bv0, bi0, jnp.float32(NEG_INF), acc0))

    pltpu.sync_copy(scores_hbm.at[w, pl.ds(NBLK * (BLK * C), TAIL * C)],
                    buf.at[pl.ds(0, TAIL * C)])
    valid = lanes < TAIL
    rid = jnp.minimum(lanes, TAIL - 1)
    mrow = _row_max(buf, rid)
    lv = labv[pl.ds(NBLK * BLK, 16)]
    fg = (lv != BG) & valid
    metric = jnp.where(fg, -mrow, NEG_INF)
    gidx = jnp.where(valid, row0 + NBLK * BLK + lanes, IMAX)
    acc = acc + jnp.where(fg, 1, 0).astype(jnp.int32)
    bv, bi, t3 = _merge_top3(metric, gidx, bv, bi)

    local = jnp.clip(bi - row0, 0, CHUNK - 1)
    labw = plsc.load_gather(labv, [local])

    ov[...] = bv
    oi[...] = bi
    ol[...] = labw
    oc[...] = acc
    pltpu.sync_copy(ov, out_v.at[w])
    pltpu.sync_copy(oi, out_i.at[w])
    pltpu.sync_copy(ol, out_lab.at[w])
    pltpu.sync_copy(oc, out_cnt.at[w])


@functools.cache
def _phase1():
    return functools.partial(
        pl.kernel,
        out_type=(
            jax.ShapeDtypeStruct((NW, 16), jnp.float32),
            jax.ShapeDtypeStruct((NW, 16), jnp.int32),
            jax.ShapeDtypeStruct((NW, 16), jnp.int32),
            jax.ShapeDtypeStruct((NW, 16), jnp.int32),
        ),
        mesh=plsc.VectorSubcoreMesh(core_axis_name="c", subcore_axis_name="s",
                                    num_cores=NC, num_subcores=NS),
        compiler_params=pltpu.CompilerParams(use_tc_tiling_on_sc=False,
                                             needs_layout_passes=False),
        scratch_types=[
            pltpu.VMEM((BLK * C,), jnp.float32),
            pltpu.VMEM((CHUNK + 6, ), jnp.int32),
            pltpu.VMEM((16,), jnp.float32),
            pltpu.VMEM((16,), jnp.int32),
            pltpu.VMEM((16,), jnp.int32),
            pltpu.VMEM((16,), jnp.int32),
        ],
    )(_phase1_body)


def _phase2_body(cv_ref, ci_ref, cl_ref, cc_ref, scores_any, out_ref,
                 r0, r1, r2, sem):
    vv = cv_ref[...]
    ii = ci_ref[...]
    labs = cl_ref[...]
    nfg = jnp.sum(cc_ref[...])

    idxs = []
    labels = []
    for _ in range(3):
        m = jnp.max(vv)
        i_k = jnp.min(jnp.where(vv == m, ii, IMAX))
        l_k = jnp.min(jnp.where(ii == i_k, labs, IMAX))
        hit = ii == i_k
        vv = jnp.where(hit, NEG_INF, vv)
        ii = jnp.where(hit, IMAX, ii)
        idxs.append(i_k)
        labels.append(l_k)

    rows = [r0, r1, r2]
    cps = []
    for j in range(3):
        cp = pltpu.make_async_copy(
            scores_any.at[pl.ds(idxs[j], 1)], rows[j], sem)
        cp.start()
        cps.append(cp)
    for cp in cps:
        cp.wait()

    col = lax.broadcasted_iota(jnp.int32, (1, C), 1)
    loss = jnp.float32(0.0)
    for j in range(3):
        row = rows[j][...]
        l_j = labels[j]
        onehot = col == l_j
        mfull = jnp.max(row)
        e = jnp.exp(row - mfull)
        gt = jnp.sum(jnp.where(onehot, e, 0.0)) / jnp.sum(e)
        t = gt * (1.0 - gt)
        masked = jnp.where(onehot, NEG_INF, row)
        mm = jnp.max(masked)
        lse = mm + jnp.log(jnp.sum(jnp.exp(masked - mm)))
        cstar = jnp.where(l_j <= C - 3, C - 2, C - 3)
        z = jnp.sum(jnp.where(col == cstar, row, 0.0)) - lse
        valid = (l_j != BG).astype(jnp.float32)
        loss = loss + (-t) * z * valid

    denom = jnp.minimum(nfg, 3).astype(jnp.float32)
    out_ref[0, 0] = loss / denom


def _phase2(cand_v, cand_i, cand_lab, cand_cnt, scores):
    return pl.pallas_call(
        _phase2_body,
        out_shape=jax.ShapeDtypeStruct((1, 1), jnp.float32),
        in_specs=[
            pl.BlockSpec(memory_space=pltpu.VMEM),
            pl.BlockSpec(memory_space=pltpu.VMEM),
            pl.BlockSpec(memory_space=pltpu.VMEM),
            pl.BlockSpec(memory_space=pltpu.VMEM),
            pl.BlockSpec(memory_space=pl.ANY),
        ],
        out_specs=pl.BlockSpec(memory_space=pltpu.SMEM),
        scratch_shapes=[
            pltpu.VMEM((1, C), jnp.float32),
            pltpu.VMEM((1, C), jnp.float32),
            pltpu.VMEM((1, C), jnp.float32),
            pltpu.SemaphoreType.DMA,
        ],
    )(cand_v, cand_i, cand_lab, cand_cnt, scores)


def kernel(scores, labels):
    scores3 = scores.reshape(NW, CHUNK * C)
    labels2 = labels.reshape(NW, CHUNK)
    cand_v, cand_i, cand_lab, cand_cnt = _phase1()(scores3, labels2)
    loss = _phase2(cand_v, cand_i, cand_lab, cand_cnt, scores)
    return loss[0, 0]


# speedup vs baseline: 2.9713x; 2.9713x over previous
import functools

import jax
import jax.numpy as jnp
from jax import lax
from jax.experimental import pallas as pl
from jax.experimental.pallas import tpu as pltpu
from jax.experimental.pallas import tpu_sc as plsc

N = 200000
C = 81
BG = 80
EXCL = 79
NC = 2
NS = 16
NW = NC * NS
CHUNK = N // NW
SPAN = 6256
BLK = 368
NBLK = SPAN // BLK
GRP = BLK // 16
IMAX = 2**31 - 1
NEG_INF = float("-inf")


def _merge_top3(cv, ci, bv, bi):
    lanes = lax.iota(jnp.int32, 16)
    nv = jnp.full((16,), NEG_INF, jnp.float32)
    ni = jnp.full((16,), IMAX, jnp.int32)
    av, ai = bv, bi
    t3 = NEG_INF
    for k in range(3):
        m = jnp.maximum(jnp.max(av), jnp.max(cv))
        ia = jnp.min(jnp.where(av == m, ai, IMAX))
        ic = jnp.min(jnp.where(cv == m, ci, IMAX))
        i = jnp.minimum(ia, ic)
        nv = jnp.where(lanes == k, m, nv)
        ni = jnp.where(lanes == k, i, ni)
        hit_a = ai == i
        av = jnp.where(hit_a, NEG_INF, av)
        ai = jnp.where(hit_a, IMAX, ai)
        hit_c = ci == i
        cv = jnp.where(hit_c, NEG_INF, cv)
        ci = jnp.where(hit_c, IMAX, ci)
        t3 = m
    return nv, ni, t3


def _row_max(buf, rid):
    m = jnp.full((16,), NEG_INF, jnp.float32)
    for c in range(C):
        if c == EXCL:
            continue
        col = jnp.full((16,), c, jnp.int32)
        m = jnp.maximum(m, plsc.load_gather(buf, [rid, col]))
    return m


def _phase1_body(scores_hbm, labels_hbm, out_v, out_i, out_lab, out_cnt,
                 buf, labv, ov, oi, ol, oc):
    w = lax.axis_index("s") * NC + lax.axis_index("c")
    lo = w * CHUNK
    astart = pl.multiple_of(lo - (lo % 8), 8)
    lanes = lax.iota(jnp.int32, 16)

    pltpu.sync_copy(labels_hbm.at[pl.ds(astart, SPAN)], labv)

    def group_step(g_off, carry):
        bv, bi, t3, acc, blk_base = carry
        rid = g_off + lanes
        mrow = _row_max(buf, rid)
        lv = labv[pl.ds(blk_base + g_off, 16)]
        gidx = astart + blk_base + rid
        win = (gidx >= lo) & (gidx < lo + CHUNK)
        fg = (lv != BG) & win
        metric = jnp.where(fg, -mrow, NEG_INF)
        gidx_m = jnp.where(win, gidx, IMAX)
        acc = acc + jnp.where(fg, 1, 0).astype(jnp.int32)
        gmax = jnp.max(metric)

        def do_merge(args):
            bv_, bi_, _ = args
            return _merge_top3(metric, gidx_m, bv_, bi_)

        bv, bi, t3 = lax.cond(gmax >= t3, do_merge, lambda a: a, (bv, bi, t3))
        return bv, bi, t3, acc, blk_base

    def block_step(b, carry):
        bv, bi, t3, acc = carry
        pltpu.sync_copy(scores_hbm.at[pl.ds(astart + b * BLK, BLK)], buf)
        blk_base = b * BLK

        def grp(g, c):
            return group_step(g * 16, c)

        bv, bi, t3, acc, _ = lax.fori_loop(
            0, GRP, grp, (bv, bi, t3, acc, blk_base))
        return bv, bi, t3, acc

    bv0 = jnp.full((16,), NEG_INF, jnp.float32)
    bi0 = jnp.full((16,), IMAX, jnp.int32)
    acc0 = jnp.zeros((16,), jnp.int32)
    bv, bi, t3, acc = lax.fori_loop(
        0, NBLK, block_step, (bv0, bi0, jnp.float32(NEG_INF), acc0))

    local = jnp.clip(bi - astart, 0, SPAN - 1)
    labw = plsc.load_gather(labv, [local])

    ov[...] = bv
    oi[...] = bi
    ol[...] = labw
    oc[...] = acc
    pltpu.sync_copy(ov, out_v.at[pl.ds(w * 16, 16)])
    pltpu.sync_copy(oi, out_i.at[pl.ds(w * 16, 16)])
    pltpu.sync_copy(ol, out_lab.at[pl.ds(w * 16, 16)])
    pltpu.sync_copy(oc, out_cnt.at[pl.ds(w * 16, 16)])


@functools.cache
def _phase1():
    return functools.partial(
        pl.kernel,
        out_type=(
            jax.ShapeDtypeStruct((NW * 16,), jnp.float32),
            jax.ShapeDtypeStruct((NW * 16,), jnp.int32),
            jax.ShapeDtypeStruct((NW * 16,), jnp.int32),
            jax.ShapeDtypeStruct((NW * 16,), jnp.int32),
        ),
        mesh=plsc.VectorSubcoreMesh(core_axis_name="c", subcore_axis_name="s",
                                    num_cores=NC, num_subcores=NS),
        compiler_params=pltpu.CompilerParams(use_tc_tiling_on_sc=True,
                                             needs_layout_passes=False),
        scratch_types=[
            pltpu.VMEM((BLK, C), jnp.float32),
            pltpu.VMEM((SPAN,), jnp.int32),
            pltpu.VMEM((16,), jnp.float32),
            pltpu.VMEM((16,), jnp.int32),
            pltpu.VMEM((16,), jnp.int32),
            pltpu.VMEM((16,), jnp.int32),
        ],
    )(_phase1_body)


def _phase2_body(cv_ref, ci_ref, cl_ref, cc_ref, scores_any, out_ref,
                 r0, r1, r2, sem):
    vv = cv_ref[...]
    ii = ci_ref[...]
    labs = cl_ref[...]
    nfg = jnp.sum(cc_ref[...])

    idxs = []
    labels = []
    for _ in range(3):
        m = jnp.max(vv)
        i_k = jnp.min(jnp.where(vv == m, ii, IMAX))
        l_k = jnp.min(jnp.where(ii == i_k, labs, IMAX))
        hit = ii == i_k
        vv = jnp.where(hit, NEG_INF, vv)
        ii = jnp.where(hit, IMAX, ii)
        idxs.append(i_k)
        labels.append(l_k)

    rows = [r0, r1, r2]
    cps = []
    for j in range(3):
        cp = pltpu.make_async_copy(
            scores_any.at[pl.ds(idxs[j], 1)], rows[j], sem)
        cp.start()
        cps.append(cp)
    for cp in cps:
        cp.wait()

    col = lax.broadcasted_iota(jnp.int32, (1, C), 1)
    loss = jnp.float32(0.0)
    for j in range(3):
        row = rows[j][...]
        l_j = labels[j]
        onehot = col == l_j
        mfull = jnp.max(row)
        e = jnp.exp(row - mfull)
        gt = jnp.sum(jnp.where(onehot, e, 0.0)) / jnp.sum(e)
        t = gt * (1.0 - gt)
        masked = jnp.where(onehot, NEG_INF, row)
        mm = jnp.max(masked)
        lse = mm + jnp.log(jnp.sum(jnp.exp(masked - mm)))
        cstar = jnp.where(l_j <= C - 3, C - 2, C - 3)
        z = jnp.sum(jnp.where(col == cstar, row, 0.0)) - lse
        valid = (l_j != BG).astype(jnp.float32)
        loss = loss + (-t) * z * valid

    denom = jnp.minimum(nfg, 3).astype(jnp.float32)
    out_ref[0, 0] = loss / denom


def _phase2(cand_v, cand_i, cand_lab, cand_cnt, scores):
    return pl.pallas_call(
        _phase2_body,
        out_shape=jax.ShapeDtypeStruct((1, 1), jnp.float32),
        in_specs=[
            pl.BlockSpec(memory_space=pltpu.VMEM),
            pl.BlockSpec(memory_space=pltpu.VMEM),
            pl.BlockSpec(memory_space=pltpu.VMEM),
            pl.BlockSpec(memory_space=pltpu.VMEM),
            pl.BlockSpec(memory_space=pl.ANY),
        ],
        out_specs=pl.BlockSpec(memory_space=pltpu.SMEM),
        scratch_shapes=[
            pltpu.VMEM((1, C), jnp.float32),
            pltpu.VMEM((1, C), jnp.float32),
            pltpu.VMEM((1, C), jnp.float32),
            pltpu.SemaphoreType.DMA,
        ],
    )(cand_v, cand_i, cand_lab, cand_cnt, scores)


def kernel(scores, labels):
    cand_v, cand_i, cand_lab, cand_cnt = _phase1()(scores, labels)
    loss = _phase2(cand_v, cand_i, cand_lab, cand_cnt, scores)
    return loss[0, 0]


# speedup vs baseline: 12.2339x; 4.1174x over previous
import functools

import jax
import jax.numpy as jnp
from jax import lax
from jax.experimental import pallas as pl
from jax.experimental.pallas import tpu as pltpu
from jax.experimental.pallas import tpu_sc as plsc

N = 200000
C = 81
BG = 80
EXCL = 79
NC = 2
NS = 16
NW = NC * NS
CHUNK = N // NW
SPAN = 6400
BLK = 640
NBLK = SPAN // BLK
GRP = BLK // 16
IMAX = 2**31 - 1
NEG_INF = float("-inf")


def _merge_top3(cv, ci, bv, bi):
    lanes = lax.iota(jnp.int32, 16)
    nv = jnp.full((16,), NEG_INF, jnp.float32)
    ni = jnp.full((16,), IMAX, jnp.int32)
    av, ai = bv, bi
    t3 = NEG_INF
    for k in range(3):
        m = jnp.maximum(jnp.max(av), jnp.max(cv))
        ia = jnp.min(jnp.where(av == m, ai, IMAX))
        ic = jnp.min(jnp.where(cv == m, ci, IMAX))
        i = jnp.minimum(ia, ic)
        nv = jnp.where(lanes == k, m, nv)
        ni = jnp.where(lanes == k, i, ni)
        hit_a = ai == i
        av = jnp.where(hit_a, NEG_INF, av)
        ai = jnp.where(hit_a, IMAX, ai)
        hit_c = ci == i
        cv = jnp.where(hit_c, NEG_INF, cv)
        ci = jnp.where(hit_c, IMAX, ci)
        t3 = m
    return nv, ni, t3


def _col_max(buf, o):
    m = jnp.full((16,), NEG_INF, jnp.float32)
    for c in range(C):
        if c == EXCL:
            continue
        m = jnp.maximum(m, buf[c, pl.ds(o, 16)])
    return m


def _phase1_body(scores_hbm, labels_hbm, out_v, out_i, out_lab, out_cnt,
                 buf, labv, ov, oi, ol, oc):
    w = lax.axis_index("s") * NC + lax.axis_index("c")
    lo = w * CHUNK
    astart = pl.multiple_of(lo - (lo % 128), 128)
    lanes = lax.iota(jnp.int32, 16)

    pltpu.sync_copy(labels_hbm.at[pl.ds(astart, SPAN)], labv)

    def group_step(g_off, carry):
        bv, bi, t3, acc, blk_base = carry
        mcls = _col_max(buf, g_off)
        lv = labv[pl.ds(blk_base + g_off, 16)]
        gidx = astart + blk_base + g_off + lanes
        win = (gidx >= lo) & (gidx < lo + CHUNK)
        fg = (lv != BG) & win
        metric = jnp.where(fg, -mcls, NEG_INF)
        gidx_m = jnp.where(win, gidx, IMAX)
        acc = acc + jnp.where(fg, 1, 0).astype(jnp.int32)
        gmax = jnp.max(metric)

        def do_merge(args):
            bv_, bi_, _ = args
            return _merge_top3(metric, gidx_m, bv_, bi_)

        bv, bi, t3 = lax.cond(gmax >= t3, do_merge, lambda a: a, (bv, bi, t3))
        return bv, bi, t3, acc, blk_base

    def block_step(b, carry):
        bv, bi, t3, acc = carry
        pltpu.sync_copy(scores_hbm.at[:, pl.ds(astart + b * BLK, BLK)], buf)
        blk_base = b * BLK

        def grp(g, c):
            return group_step(g * 16, c)

        bv, bi, t3, acc, _ = lax.fori_loop(
            0, GRP, grp, (bv, bi, t3, acc, blk_base))
        return bv, bi, t3, acc

    bv0 = jnp.full((16,), NEG_INF, jnp.float32)
    bi0 = jnp.full((16,), IMAX, jnp.int32)
    acc0 = jnp.zeros((16,), jnp.int32)
    bv, bi, t3, acc = lax.fori_loop(
        0, NBLK, block_step, (bv0, bi0, jnp.float32(NEG_INF), acc0))

    local = jnp.clip(bi - astart, 0, SPAN - 1)
    labw = plsc.load_gather(labv, [local])

    ov[...] = bv
    oi[...] = bi
    ol[...] = labw
    oc[...] = acc
    pltpu.sync_copy(ov, out_v.at[pl.ds(w * 16, 16)])
    pltpu.sync_copy(oi, out_i.at[pl.ds(w * 16, 16)])
    pltpu.sync_copy(ol, out_lab.at[pl.ds(w * 16, 16)])
    pltpu.sync_copy(oc, out_cnt.at[pl.ds(w * 16, 16)])


@functools.cache
def _phase1():
    return functools.partial(
        pl.kernel,
        out_type=(
            jax.ShapeDtypeStruct((NW * 16,), jnp.float32),
            jax.ShapeDtypeStruct((NW * 16,), jnp.int32),
            jax.ShapeDtypeStruct((NW * 16,), jnp.int32),
            jax.ShapeDtypeStruct((NW * 16,), jnp.int32),
        ),
        mesh=plsc.VectorSubcoreMesh(core_axis_name="c", subcore_axis_name="s",
                                    num_cores=NC, num_subcores=NS),
        compiler_params=pltpu.CompilerParams(use_tc_tiling_on_sc=True,
                                             needs_layout_passes=False),
        scratch_types=[
            pltpu.VMEM((C, BLK), jnp.float32),
            pltpu.VMEM((SPAN,), jnp.int32),
            pltpu.VMEM((16,), jnp.float32),
            pltpu.VMEM((16,), jnp.int32),
            pltpu.VMEM((16,), jnp.int32),
            pltpu.VMEM((16,), jnp.int32),
        ],
    )(_phase1_body)


def _phase2_body(cv_ref, ci_ref, cl_ref, cc_ref, scores_any, out_ref,
                 r0, r1, r2, sem):
    vv = cv_ref[...]
    ii = ci_ref[...]
    labs = cl_ref[...]
    nfg = jnp.sum(cc_ref[...])

    idxs = []
    labels = []
    for _ in range(3):
        m = jnp.max(vv)
        i_k = jnp.min(jnp.where(vv == m, ii, IMAX))
        l_k = jnp.min(jnp.where(ii == i_k, labs, IMAX))
        hit = ii == i_k
        vv = jnp.where(hit, NEG_INF, vv)
        ii = jnp.where(hit, IMAX, ii)
        idxs.append(i_k)
        labels.append(l_k)

    bufs = [r0, r1, r2]
    cps = []
    bases = []
    for j in range(3):
        base = pl.multiple_of(idxs[j] - lax.rem(idxs[j], 128), 128)
        bases.append(base)
        cp = pltpu.make_async_copy(
            scores_any.at[:, pl.ds(base, 128)], bufs[j], sem)
        cp.start()
        cps.append(cp)
    for cp in cps:
        cp.wait()

    col = lax.broadcasted_iota(jnp.int32, (C, 1), 0)
    lane = lax.broadcasted_iota(jnp.int32, (C, 128), 1)
    loss = jnp.float32(0.0)
    for j in range(3):
        d = idxs[j] - bases[j]
        row = jnp.sum(jnp.where(lane == d, bufs[j][...], 0.0),
                      axis=1, keepdims=True)
        l_j = labels[j]
        onehot = col == l_j
        mfull = jnp.max(row)
        e = jnp.exp(row - mfull)
        gt = jnp.sum(jnp.where(onehot, e, 0.0)) / jnp.sum(e)
        t = gt * (1.0 - gt)
        masked = jnp.where(onehot, NEG_INF, row)
        mm = jnp.max(masked)
        lse = mm + jnp.log(jnp.sum(jnp.exp(masked - mm)))
        cstar = jnp.where(l_j <= C - 3, C - 2, C - 3)
        z = jnp.sum(jnp.where(col == cstar, row, 0.0)) - lse
        valid = (l_j != BG).astype(jnp.float32)
        loss = loss + (-t) * z * valid

    denom = jnp.minimum(nfg, 3).astype(jnp.float32)
    out_ref[0, 0] = loss / denom


def _phase2(cand_v, cand_i, cand_lab, cand_cnt, scores_t):
    return pl.pallas_call(
        _phase2_body,
        out_shape=jax.ShapeDtypeStruct((1, 1), jnp.float32),
        in_specs=[
            pl.BlockSpec(memory_space=pltpu.VMEM),
            pl.BlockSpec(memory_space=pltpu.VMEM),
            pl.BlockSpec(memory_space=pltpu.VMEM),
            pl.BlockSpec(memory_space=pltpu.VMEM),
            pl.BlockSpec(memory_space=pl.ANY),
        ],
        out_specs=pl.BlockSpec(memory_space=pltpu.SMEM),
        scratch_shapes=[
            pltpu.VMEM((C, 128), jnp.float32),
            pltpu.VMEM((C, 128), jnp.float32),
            pltpu.VMEM((C, 128), jnp.float32),
            pltpu.SemaphoreType.DMA,
        ],
    )(cand_v, cand_i, cand_lab, cand_cnt, scores_t)


def kernel(scores, labels):
    scores_t = scores.T
    cand_v, cand_i, cand_lab, cand_cnt = _phase1()(scores_t, labels)
    loss = _phase2(cand_v, cand_i, cand_lab, cand_cnt, scores_t)
    return loss[0, 0]


# speedup vs baseline: 17.7073x; 1.4474x over previous
import functools

import jax
import jax.numpy as jnp
from jax import lax
from jax.experimental import pallas as pl
from jax.experimental.pallas import tpu as pltpu
from jax.experimental.pallas import tpu_sc as plsc

N = 200000
C = 81
BG = 80
EXCL = 79
NC = 2
NS = 16
NW = NC * NS
CHUNK = N // NW
SPAN = 6400
BLK = 640
NBLK = SPAN // BLK
GRP = BLK // 16
IMAX = 2**31 - 1
NEG_INF = float("-inf")


def _merge_top3(cv, ci, bv, bi):
    lanes = lax.iota(jnp.int32, 16)
    nv = jnp.full((16,), NEG_INF, jnp.float32)
    ni = jnp.full((16,), IMAX, jnp.int32)
    av, ai = bv, bi
    t3 = NEG_INF
    for k in range(3):
        m = jnp.maximum(jnp.max(av), jnp.max(cv))
        ia = jnp.min(jnp.where(av == m, ai, IMAX))
        ic = jnp.min(jnp.where(cv == m, ci, IMAX))
        i = jnp.minimum(ia, ic)
        nv = jnp.where(lanes == k, m, nv)
        ni = jnp.where(lanes == k, i, ni)
        hit_a = ai == i
        av = jnp.where(hit_a, NEG_INF, av)
        ai = jnp.where(hit_a, IMAX, ai)
        hit_c = ci == i
        cv = jnp.where(hit_c, NEG_INF, cv)
        ci = jnp.where(hit_c, IMAX, ci)
        t3 = m
    return nv, ni, t3


def _col_max(buf, o):
    m = jnp.full((16,), NEG_INF, jnp.float32)
    for c in range(C):
        if c == EXCL:
            continue
        m = jnp.maximum(m, buf[c, pl.ds(o, 16)])
    return m


def _phase1_body(scores_hbm, labels_hbm, out_v, out_i, out_lab, out_cnt,
                 buf0, buf1, labv, ov, oi, ol, oc, sem0, sem1):
    w = lax.axis_index("s") * NC + lax.axis_index("c")
    lo = w * CHUNK
    astart = pl.multiple_of(lo - (lo % 128), 128)
    lanes = lax.iota(jnp.int32, 16)

    def start_blk(b, buf, sem):
        pltpu.async_copy(scores_hbm.at[:, pl.ds(astart + b * BLK, BLK)],
                         buf, sem)

    def wait_blk(b, buf, sem):
        pltpu.make_async_copy(scores_hbm.at[:, pl.ds(astart + b * BLK, BLK)],
                              buf, sem).wait()

    start_blk(0, buf0, sem0)
    pltpu.sync_copy(labels_hbm.at[pl.ds(astart, SPAN)], labv)

    def group_step(buf, g_off, carry):
        bv, bi, t3, acc, blk_base = carry
        mcls = _col_max(buf, g_off)
        lv = labv[pl.ds(blk_base + g_off, 16)]
        gidx = astart + blk_base + g_off + lanes
        win = (gidx >= lo) & (gidx < lo + CHUNK)
        fg = (lv != BG) & win
        metric = jnp.where(fg, -mcls, NEG_INF)
        gidx_m = jnp.where(win, gidx, IMAX)
        acc = acc + jnp.where(fg, 1, 0).astype(jnp.int32)
        gmax = jnp.max(metric)

        def do_merge(args):
            bv_, bi_, _ = args
            return _merge_top3(metric, gidx_m, bv_, bi_)

        bv, bi, t3 = lax.cond(gmax >= t3, do_merge, lambda a: a, (bv, bi, t3))
        return bv, bi, t3, acc, blk_base

    def compute_blk(buf, b, carry):
        bv, bi, t3, acc = carry

        def grp(g, c):
            return group_step(buf, g * 16, c)

        bv, bi, t3, acc, _ = lax.fori_loop(
            0, GRP, grp, (bv, bi, t3, acc, b * BLK))
        return bv, bi, t3, acc

    def pair_step(i, carry):
        b0 = i * 2
        wait_blk(b0, buf0, sem0)
        start_blk(b0 + 1, buf1, sem1)
        carry = compute_blk(buf0, b0, carry)
        wait_blk(b0 + 1, buf1, sem1)

        @pl.when(i < NBLK // 2 - 1)
        def _():
            start_blk(b0 + 2, buf0, sem0)

        return compute_blk(buf1, b0 + 1, carry)

    bv0 = jnp.full((16,), NEG_INF, jnp.float32)
    bi0 = jnp.full((16,), IMAX, jnp.int32)
    acc0 = jnp.zeros((16,), jnp.int32)
    bv, bi, t3, acc = lax.fori_loop(
        0, NBLK // 2, pair_step, (bv0, bi0, jnp.float32(NEG_INF), acc0))

    local = jnp.clip(bi - astart, 0, SPAN - 1)
    labw = plsc.load_gather(labv, [local])

    ov[...] = bv
    oi[...] = bi
    ol[...] = labw
    oc[...] = acc
    pltpu.sync_copy(ov, out_v.at[pl.ds(w * 16, 16)])
    pltpu.sync_copy(oi, out_i.at[pl.ds(w * 16, 16)])
    pltpu.sync_copy(ol, out_lab.at[pl.ds(w * 16, 16)])
    pltpu.sync_copy(oc, out_cnt.at[pl.ds(w * 16, 16)])


@functools.cache
def _phase1():
    return functools.partial(
        pl.kernel,
        out_type=(
            jax.ShapeDtypeStruct((NW * 16,), jnp.float32),
            jax.ShapeDtypeStruct((NW * 16,), jnp.int32),
            jax.ShapeDtypeStruct((NW * 16,), jnp.int32),
            jax.ShapeDtypeStruct((NW * 16,), jnp.int32),
        ),
        mesh=plsc.VectorSubcoreMesh(core_axis_name="c", subcore_axis_name="s",
                                    num_cores=NC, num_subcores=NS),
        compiler_params=pltpu.CompilerParams(use_tc_tiling_on_sc=True,
                                             needs_layout_passes=False),
        scratch_types=[
            pltpu.VMEM((C, BLK), jnp.float32),
            pltpu.VMEM((C, BLK), jnp.float32),
            pltpu.VMEM((SPAN,), jnp.int32),
            pltpu.VMEM((16,), jnp.float32),
            pltpu.VMEM((16,), jnp.int32),
            pltpu.VMEM((16,), jnp.int32),
            pltpu.VMEM((16,), jnp.int32),
            pltpu.SemaphoreType.DMA,
            pltpu.SemaphoreType.DMA,
        ],
    )(_phase1_body)


def _phase2_body(cv_ref, ci_ref, cl_ref, cc_ref, scores_any, out_ref,
                 r0, r1, r2, sem):
    vv = cv_ref[...]
    ii = ci_ref[...]
    labs = cl_ref[...]
    nfg = jnp.sum(cc_ref[...])

    idxs = []
    labels = []
    for _ in range(3):
        m = jnp.max(vv)
        i_k = jnp.min(jnp.where(vv == m, ii, IMAX))
        l_k = jnp.min(jnp.where(ii == i_k, labs, IMAX))
        hit = ii == i_k
        vv = jnp.where(hit, NEG_INF, vv)
        ii = jnp.where(hit, IMAX, ii)
        idxs.append(i_k)
        labels.append(l_k)

    bufs = [r0, r1, r2]
    cps = []
    bases = []
    for j in range(3):
        base = pl.multiple_of(idxs[j] - lax.rem(idxs[j], 128), 128)
        bases.append(base)
        cp = pltpu.make_async_copy(
            scores_any.at[:, pl.ds(base, 128)], bufs[j], sem)
        cp.start()
        cps.append(cp)
    for cp in cps:
        cp.wait()

    col = lax.broadcasted_iota(jnp.int32, (C, 1), 0)
    lane = lax.broadcasted_iota(jnp.int32, (C, 128), 1)
    loss = jnp.float32(0.0)
    for j in range(3):
        d = idxs[j] - bases[j]
        row = jnp.sum(jnp.where(lane == d, bufs[j][...], 0.0),
                      axis=1, keepdims=True)
        l_j = labels[j]
        onehot = col == l_j
        mfull = jnp.max(row)
        e = jnp.exp(row - mfull)
        gt = jnp.sum(jnp.where(onehot, e, 0.0)) / jnp.sum(e)
        t = gt * (1.0 - gt)
        masked = jnp.where(onehot, NEG_INF, row)
        mm = jnp.max(masked)
        lse = mm + jnp.log(jnp.sum(jnp.exp(masked - mm)))
        cstar = jnp.where(l_j <= C - 3, C - 2, C - 3)
        z = jnp.sum(jnp.where(col == cstar, row, 0.0)) - lse
        valid = (l_j != BG).astype(jnp.float32)
        loss = loss + (-t) * z * valid

    denom = jnp.minimum(nfg, 3).astype(jnp.float32)
    out_ref[0, 0] = loss / denom


def _phase2(cand_v, cand_i, cand_lab, cand_cnt, scores_t):
    return pl.pallas_call(
        _phase2_body,
        out_shape=jax.ShapeDtypeStruct((1, 1), jnp.float32),
        in_specs=[
            pl.BlockSpec(memory_space=pltpu.VMEM),
            pl.BlockSpec(memory_space=pltpu.VMEM),
            pl.BlockSpec(memory_space=pltpu.VMEM),
            pl.BlockSpec(memory_space=pltpu.VMEM),
            pl.BlockSpec(memory_space=pl.ANY),
        ],
        out_specs=pl.BlockSpec(memory_space=pltpu.SMEM),
        scratch_shapes=[
            pltpu.VMEM((C, 128), jnp.float32),
            pltpu.VMEM((C, 128), jnp.float32),
            pltpu.VMEM((C, 128), jnp.float32),
            pltpu.SemaphoreType.DMA,
        ],
    )(cand_v, cand_i, cand_lab, cand_cnt, scores_t)


def kernel(scores, labels):
    scores_t = scores.T
    cand_v, cand_i, cand_lab, cand_cnt = _phase1()(scores_t, labels)
    loss = _phase2(cand_v, cand_i, cand_lab, cand_cnt, scores_t)
    return loss[0, 0]


# speedup vs baseline: 18.0813x; 1.0211x over previous
import functools

import jax
import jax.numpy as jnp
from jax import lax
from jax.experimental import pallas as pl
from jax.experimental.pallas import tpu as pltpu
from jax.experimental.pallas import tpu_sc as plsc

N = 200000
C = 81
BG = 80
EXCL = 79
NC = 2
NS = 16
NW = NC * NS
CHUNK = N // NW
SPAN = 6400
BLK = 640
NBLK = SPAN // BLK
GRP = BLK // 16
IMAX = 2**31 - 1
NEG_INF = float("-inf")


def _merge_top3(cv, ci, bv, bi):
    lanes = lax.iota(jnp.int32, 16)
    nv = jnp.full((16,), NEG_INF, jnp.float32)
    ni = jnp.full((16,), IMAX, jnp.int32)
    av, ai = bv, bi
    t3 = NEG_INF
    for k in range(3):
        m = jnp.maximum(jnp.max(av), jnp.max(cv))
        ia = jnp.min(jnp.where(av == m, ai, IMAX))
        ic = jnp.min(jnp.where(cv == m, ci, IMAX))
        i = jnp.minimum(ia, ic)
        nv = jnp.where(lanes == k, m, nv)
        ni = jnp.where(lanes == k, i, ni)
        hit_a = ai == i
        av = jnp.where(hit_a, NEG_INF, av)
        ai = jnp.where(hit_a, IMAX, ai)
        hit_c = ci == i
        cv = jnp.where(hit_c, NEG_INF, cv)
        ci = jnp.where(hit_c, IMAX, ci)
        t3 = m
    return nv, ni, t3


def _col_max(buf, o):
    m = jnp.full((16,), NEG_INF, jnp.float32)
    for c in range(C):
        if c == EXCL:
            continue
        m = jnp.maximum(m, buf[c, pl.ds(o, 16)])
    return m


def _phase1_body(scores_hbm, labels_hbm, out_v, out_i, out_lab, out_cnt,
                 buf0, buf1, labv, ov, oi, ol, oc, sem0, sem1):
    w = lax.axis_index("s") * NC + lax.axis_index("c")
    lo = w * CHUNK
    astart = pl.multiple_of(lo - (lo % 128), 128)
    lanes = lax.iota(jnp.int32, 16)

    def start_blk(b, buf, sem):
        pltpu.async_copy(scores_hbm.at[:, pl.ds(astart + b * BLK, BLK)],
                         buf, sem)

    def wait_blk(b, buf, sem):
        pltpu.make_async_copy(scores_hbm.at[:, pl.ds(astart + b * BLK, BLK)],
                              buf, sem).wait()

    start_blk(0, buf0, sem0)
    pltpu.sync_copy(labels_hbm.at[pl.ds(astart, SPAN)], labv)

    def half(buf, g_off, blk_base):
        mcls = _col_max(buf, g_off)
        lv = labv[pl.ds(blk_base + g_off, 16)]
        gidx = astart + blk_base + g_off + lanes
        win = (gidx >= lo) & (gidx < lo + CHUNK)
        fg = (lv != BG) & win
        metric = jnp.where(fg, -mcls, NEG_INF)
        gidx_m = jnp.where(win, gidx, IMAX)
        return metric, gidx_m, fg

    def group_step(buf, g_off, carry):
        v1, i1, v2, i2, v3, i3, acc, blk_base = carry
        metric, gidx, fg = half(buf, g_off, blk_base)
        acc = acc + jnp.where(fg, 1, 0).astype(jnp.int32)
        gt1 = metric > v1
        gt2 = metric > v2
        gt3 = metric > v3
        v3 = jnp.where(gt2, v2, jnp.where(gt3, metric, v3))
        i3 = jnp.where(gt2, i2, jnp.where(gt3, gidx, i3))
        v2 = jnp.where(gt1, v1, jnp.where(gt2, metric, v2))
        i2 = jnp.where(gt1, i1, jnp.where(gt2, gidx, i2))
        v1 = jnp.where(gt1, metric, v1)
        i1 = jnp.where(gt1, gidx, i1)
        return v1, i1, v2, i2, v3, i3, acc, blk_base

    def compute_blk(buf, b, carry):
        def grp(g, c):
            return group_step(buf, g * 16, c)

        out = lax.fori_loop(0, GRP, grp, carry[:7] + (b * BLK,))
        return out[:7]

    def pair_step(i, carry):
        b0 = i * 2
        wait_blk(b0, buf0, sem0)
        start_blk(b0 + 1, buf1, sem1)
        carry = compute_blk(buf0, b0, carry)
        wait_blk(b0 + 1, buf1, sem1)

        @pl.when(i < NBLK // 2 - 1)
        def _():
            start_blk(b0 + 2, buf0, sem0)

        return compute_blk(buf1, b0 + 1, carry)

    nv0 = jnp.full((16,), NEG_INF, jnp.float32)
    ni0 = jnp.full((16,), IMAX, jnp.int32)
    acc0 = jnp.zeros((16,), jnp.int32)
    v1, i1, v2, i2, v3, i3, acc = lax.fori_loop(
        0, NBLK // 2, pair_step, (nv0, ni0, nv0, ni0, nv0, ni0, acc0))

    bv, bi, _ = _merge_top3(v1, i1, nv0, ni0)
    bv, bi, _ = _merge_top3(v2, i2, bv, bi)
    bv, bi, _ = _merge_top3(v3, i3, bv, bi)

    local = jnp.clip(bi - astart, 0, SPAN - 1)
    labw = plsc.load_gather(labv, [local])

    ov[...] = bv
    oi[...] = bi
    ol[...] = labw
    oc[...] = acc
    pltpu.sync_copy(ov, out_v.at[pl.ds(w * 16, 16)])
    pltpu.sync_copy(oi, out_i.at[pl.ds(w * 16, 16)])
    pltpu.sync_copy(ol, out_lab.at[pl.ds(w * 16, 16)])
    pltpu.sync_copy(oc, out_cnt.at[pl.ds(w * 16, 16)])


@functools.cache
def _phase1():
    return functools.partial(
        pl.kernel,
        out_type=(
            jax.ShapeDtypeStruct((NW * 16,), jnp.float32),
            jax.ShapeDtypeStruct((NW * 16,), jnp.int32),
            jax.ShapeDtypeStruct((NW * 16,), jnp.int32),
            jax.ShapeDtypeStruct((NW * 16,), jnp.int32),
        ),
        mesh=plsc.VectorSubcoreMesh(core_axis_name="c", subcore_axis_name="s",
                                    num_cores=NC, num_subcores=NS),
        compiler_params=pltpu.CompilerParams(use_tc_tiling_on_sc=True,
                                             needs_layout_passes=False),
        scratch_types=[
            pltpu.VMEM((C, BLK), jnp.float32),
            pltpu.VMEM((C, BLK), jnp.float32),
            pltpu.VMEM((SPAN,), jnp.int32),
            pltpu.VMEM((16,), jnp.float32),
            pltpu.VMEM((16,), jnp.int32),
            pltpu.VMEM((16,), jnp.int32),
            pltpu.VMEM((16,), jnp.int32),
            pltpu.SemaphoreType.DMA,
            pltpu.SemaphoreType.DMA,
        ],
    )(_phase1_body)


def _phase2_body(cv_ref, ci_ref, cl_ref, cc_ref, scores_any, out_ref,
                 r0, r1, r2, sem):
    vv = cv_ref[...]
    ii = ci_ref[...]
    labs = cl_ref[...]
    nfg = jnp.sum(cc_ref[...])

    idxs = []
    labels = []
    for _ in range(3):
        m = jnp.max(vv)
        i_k = jnp.min(jnp.where(vv == m, ii, IMAX))
        l_k = jnp.min(jnp.where(ii == i_k, labs, IMAX))
        hit = ii == i_k
        vv = jnp.where(hit, NEG_INF, vv)
        ii = jnp.where(hit, IMAX, ii)
        idxs.append(i_k)
        labels.append(l_k)

    bufs = [r0, r1, r2]
    cps = []
    bases = []
    for j in range(3):
        base = pl.multiple_of(idxs[j] - lax.rem(idxs[j], 128), 128)
        bases.append(base)
        cp = pltpu.make_async_copy(
            scores_any.at[:, pl.ds(base, 128)], bufs[j], sem)
        cp.start()
        cps.append(cp)
    for cp in cps:
        cp.wait()

    col = lax.broadcasted_iota(jnp.int32, (C, 1), 0)
    lane = lax.broadcasted_iota(jnp.int32, (C, 128), 1)
    loss = jnp.float32(0.0)
    for j in range(3):
        d = idxs[j] - bases[j]
        row = jnp.sum(jnp.where(lane == d, bufs[j][...], 0.0),
                      axis=1, keepdims=True)
        l_j = labels[j]
        onehot = col == l_j
        mfull = jnp.max(row)
        e = jnp.exp(row - mfull)
        gt = jnp.sum(jnp.where(onehot, e, 0.0)) / jnp.sum(e)
        t = gt * (1.0 - gt)
        masked = jnp.where(onehot, NEG_INF, row)
        mm = jnp.max(masked)
        lse = mm + jnp.log(jnp.sum(jnp.exp(masked - mm)))
        cstar = jnp.where(l_j <= C - 3, C - 2, C - 3)
        z = jnp.sum(jnp.where(col == cstar, row, 0.0)) - lse
        valid = (l_j != BG).astype(jnp.float32)
        loss = loss + (-t) * z * valid

    denom = jnp.minimum(nfg, 3).astype(jnp.float32)
    out_ref[0, 0] = loss / denom


def _phase2(cand_v, cand_i, cand_lab, cand_cnt, scores_t):
    return pl.pallas_call(
        _phase2_body,
        out_shape=jax.ShapeDtypeStruct((1, 1), jnp.float32),
        in_specs=[
            pl.BlockSpec(memory_space=pltpu.VMEM),
            pl.BlockSpec(memory_space=pltpu.VMEM),
            pl.BlockSpec(memory_space=pltpu.VMEM),
            pl.BlockSpec(memory_space=pltpu.VMEM),
            pl.BlockSpec(memory_space=pl.ANY),
        ],
        out_specs=pl.BlockSpec(memory_space=pltpu.SMEM),
        scratch_shapes=[
            pltpu.VMEM((C, 128), jnp.float32),
            pltpu.VMEM((C, 128), jnp.float32),
            pltpu.VMEM((C, 128), jnp.float32),
            pltpu.SemaphoreType.DMA,
        ],
    )(cand_v, cand_i, cand_lab, cand_cnt, scores_t)


def kernel(scores, labels):
    scores_t = scores.T
    cand_v, cand_i, cand_lab, cand_cnt = _phase1()(scores_t, labels)
    loss = _phase2(cand_v, cand_i, cand_lab, cand_cnt, scores_t)
    return loss[0, 0]
